# baseline (speedup 1.0000x reference)
import functools

import jax
import jax.numpy as jnp
from jax import lax
from jax.experimental import pallas as pl
from jax.experimental.pallas import tpu as pltpu
from jax.experimental.pallas import tpu_sc as plsc

N_LABELS = 16384
VOCAB = 17
D = 128

NS = 16
L = 16

PER_W = N_LABELS // NS
NVEC = PER_W // L
NCOL = D // L


@functools.partial(
    pl.kernel,
    out_type=jax.ShapeDtypeStruct((NCOL, L), jnp.float32),
    mesh=plsc.VectorSubcoreMesh(
        core_axis_name="c", subcore_axis_name="s", num_cores=1, num_subcores=NS
    ),
    compiler_params=pltpu.CompilerParams(needs_layout_passes=False),
    scratch_types=[
        pltpu.VMEM((PER_W,), jnp.int32),
        pltpu.VMEM((D,), jnp.float32),
        pltpu.VMEM((NS, D), jnp.float32),
        pltpu.VMEM((VOCAB, D), jnp.float32),
        pltpu.VMEM((D, D), jnp.float32),
        pltpu.VMEM((L,), jnp.float32),
        pltpu.VMEM_SHARED((NS, D), jnp.float32),
        pltpu.SemaphoreType.DMA,
        pltpu.SemaphoreType.DMA,
    ],
)
def _sc_contract(idx_hbm, table_hbm, w_hbm, out_hbm,
                 idx_v, hist_v, hists_v, table_v, w_v, outst_v,
                 sh_hists, tsem, wsem):
    s = lax.axis_index("s")
    base = s * PER_W

    zeros = jnp.zeros((L,), jnp.float32)

    @pl.when(s < NCOL)
    def _prefetch():
        pltpu.async_copy(table_hbm, table_v, tsem)
        pltpu.async_copy(w_hbm, w_v, wsem)

    hist_v[pl.ds(0, L)] = zeros
    hist_v[pl.ds(L, L)] = zeros
    pltpu.sync_copy(idx_hbm.at[pl.ds(base, PER_W)], idx_v)
    ones = jnp.full((L,), 1.0 / N_LABELS, jnp.float32)
    for i in range(NVEC):
        iv = idx_v[pl.ds(i * L, L)]
        plsc.addupdate_scatter(hist_v, [iv], ones)
    pltpu.sync_copy(hist_v, sh_hists.at[s])

    plsc.subcore_barrier()

    @pl.when(s < NCOL)
    def _stage():
        pltpu.sync_copy(sh_hists, hists_v)
        tot0 = zeros
        tot1 = zeros
        for w in range(NS):
            tot0 = tot0 + hists_v[w, pl.ds(0, L)]
            tot1 = tot1 + hists_v[w, pl.ds(L, L)]
        w0 = [tot0[v] for v in range(L)]
        w16 = tot1[0]

        pltpu.make_async_copy(table_hbm, table_v, tsem).wait()
        pooled = []
        for kc in range(NCOL):
            acc = zeros
            for v in range(L):
                acc = acc + w0[v] * table_v[v, pl.ds(kc * L, L)]
            acc = acc + w16 * table_v[L, pl.ds(kc * L, L)]
            pooled.append(acc)

        pltpu.make_async_copy(w_hbm, w_v, wsem).wait()
        col = s * L
        acc = zeros
        for kc in range(NCOL):
            tp = pooled[kc]
            for kl in range(L):
                acc = acc + tp[kl] * w_v[kc * L + kl, pl.ds(col, L)]
        outst_v[...] = acc
        pltpu.sync_copy(outst_v, out_hbm.at[s])


def kernel(indices, table, W):
    parts = _sc_contract(indices.astype(jnp.int32), table, W)
    return parts.reshape(D)
